# Initial kernel scaffold; baseline (speedup 1.0000x reference)
import jax
import jax.numpy as jnp
from jax.experimental import pallas as pl


def kernel(x, edge_index, W1, b1, W2, b2):
    raise NotImplementedError("write your pallas kernel here")


# speedup vs baseline: 6.4739x; 6.4739x over previous
import functools

import jax
import jax.numpy as jnp
from jax import lax
from jax.experimental import pallas as pl
from jax.experimental.pallas import tpu as pltpu
from jax.experimental.pallas import tpu_sc as plsc

N = 10000
E = 320000
D = 64
DF = 128
NC = 2
NS = 16
NW = NC * NS
EPW = E // NW
CH = 80
NSTEP = EPW // CH
NBLK = 5
SPB = NSTEP // NBLK
CPR = 632
TAIL = N - (NS - 1) * CPR
W16 = 16
RB = 1000

_sc_mesh = plsc.VectorSubcoreMesh(core_axis_name="c", subcore_axis_name="s")


def _coop_rows(s, fn):
    r0 = pl.multiple_of(s * CPR, 8)

    @pl.when(s < NS - 1)
    def _():
        fn(r0, CPR)

    @pl.when(s == NS - 1)
    def _():
        fn(r0, TAIL)


@functools.partial(
    pl.kernel,
    out_type=jax.ShapeDtypeStruct((NC, N, DF), jnp.float32),
    mesh=_sc_mesh,
    scratch_types=[
        pltpu.VMEM_SHARED((N, DF), jnp.float32),
        pltpu.VMEM((SPB, CH), jnp.int32),
        pltpu.VMEM((CH, DF), jnp.float32),
    ],
)
def _dego_kernel(src4, ones_h, zeros_h, out, dg, src_t, ones_v):
    c = lax.axis_index("c")
    s = lax.axis_index("s")

    _coop_rows(s, lambda r0, nr: pltpu.sync_copy(
        zeros_h.at[pl.ds(0, nr)], dg.at[pl.ds(r0, nr)]))
    pltpu.sync_copy(ones_h, ones_v)
    plsc.subcore_barrier()

    @pl.loop(0, NSTEP)
    def _(j):
        jj = lax.rem(j, SPB)
        blk = lax.div(j, SPB)

        @pl.when(jj == 0)
        def _():
            pltpu.sync_copy(src4.at[c].at[s].at[blk], src_t)

        pltpu.sync_copy(ones_v, dg.at[src_t.at[jj]], add=True)

    plsc.subcore_barrier()
    _coop_rows(s, lambda r0, nr: pltpu.sync_copy(
        dg.at[pl.ds(r0, nr)], out.at[c].at[pl.ds(r0, nr)]))


@functools.partial(
    pl.kernel,
    out_type=jax.ShapeDtypeStruct((NC, N, DF), jnp.float32),
    mesh=_sc_mesh,
    scratch_types=[
        pltpu.VMEM_SHARED((N, DF), jnp.float32),
        pltpu.VMEM((SPB, CH), jnp.int32),
        pltpu.VMEM((SPB, CH), jnp.int32),
        pltpu.VMEM((CH, DF), jnp.float32),
        pltpu.SemaphoreType.DMA,
    ],
)
def _agg_kernel(h_hbm, src4, dst4, zeros_h, out, agg, src_t, dst_t, rows, sem):
    c = lax.axis_index("c")
    s = lax.axis_index("s")

    _coop_rows(s, lambda r0, nr: pltpu.sync_copy(
        zeros_h.at[pl.ds(0, nr)], agg.at[pl.ds(r0, nr)]))
    plsc.subcore_barrier()

    @pl.loop(0, NSTEP)
    def _(j):
        jj = lax.rem(j, SPB)
        blk = lax.div(j, SPB)

        @pl.when(jj == 0)
        def _():
            pltpu.sync_copy(src4.at[c].at[s].at[blk], src_t)
            pltpu.sync_copy(dst4.at[c].at[s].at[blk], dst_t)

        pltpu.async_copy(h_hbm.at[src_t.at[jj]], rows, sem).wait()
        pltpu.sync_copy(rows, agg.at[dst_t.at[jj]], add=True)

    plsc.subcore_barrier()
    _coop_rows(s, lambda r0, nr: pltpu.sync_copy(
        agg.at[pl.ds(r0, nr)], out.at[c].at[pl.ds(r0, nr)]))


def _norm_body(degp_ref, x_ref, ns_ref, h1_ref):
    deg_o = degp_ref[0, :, 0:1] + degp_ref[1, :, 0:1]
    ns = jnp.where(deg_o > 0, lax.rsqrt(jnp.maximum(deg_o, 1.0)), 0.0)
    ns_ref[...] = jnp.broadcast_to(ns, (RB, W16))
    h1 = x_ref[...][:, D:] * ns
    pad = jnp.concatenate(
        [jnp.ones((RB, 1), jnp.float32), jnp.zeros((RB, D - 1), jnp.float32)],
        axis=-1)
    h1_ref[...] = jnp.concatenate([h1, pad], axis=-1)


_norm = pl.pallas_call(
    _norm_body,
    grid=(N // RB,),
    in_specs=[
        pl.BlockSpec((NC, RB, DF), lambda i: (0, i, 0)),
        pl.BlockSpec((RB, 2 * D), lambda i: (i, 0)),
    ],
    out_specs=[
        pl.BlockSpec((RB, W16), lambda i: (i, 0)),
        pl.BlockSpec((RB, DF), lambda i: (i, 0)),
    ],
    out_shape=[
        jax.ShapeDtypeStruct((N, W16), jnp.float32),
        jax.ShapeDtypeStruct((N, DF), jnp.float32),
    ],
)


def _lin1_body(x_ref, p_ref, ns_ref, w_ref, b_ref, y_ref, h2_ref, nd_ref):
    deg_i = p_ref[0, :, D:D + 1] + p_ref[1, :, D:D + 1]
    nd = jnp.where(deg_i > 0, lax.rsqrt(jnp.maximum(deg_i, 1.0)), 0.0)
    nd_ref[...] = jnp.broadcast_to(nd, (RB, W16))
    agg = (p_ref[0, :, :D] + p_ref[1, :, :D]) * nd
    y = (x_ref[...][:, :D]
         + jnp.dot(agg, w_ref[...], preferred_element_type=jnp.float32)
         + b_ref[...])
    y_ref[...] = y
    h2 = y * ns_ref[...][:, :1]
    h2_ref[...] = jnp.concatenate([h2, jnp.zeros_like(h2)], axis=-1)


_lin1 = pl.pallas_call(
    _lin1_body,
    grid=(N // RB,),
    in_specs=[
        pl.BlockSpec((RB, 2 * D), lambda i: (i, 0)),
        pl.BlockSpec((NC, RB, DF), lambda i: (0, i, 0)),
        pl.BlockSpec((RB, W16), lambda i: (i, 0)),
        pl.BlockSpec((D, D), lambda i: (0, 0)),
        pl.BlockSpec((1, D), lambda i: (0, 0)),
    ],
    out_specs=[
        pl.BlockSpec((RB, D), lambda i: (i, 0)),
        pl.BlockSpec((RB, DF), lambda i: (i, 0)),
        pl.BlockSpec((RB, W16), lambda i: (i, 0)),
    ],
    out_shape=[
        jax.ShapeDtypeStruct((N, D), jnp.float32),
        jax.ShapeDtypeStruct((N, DF), jnp.float32),
        jax.ShapeDtypeStruct((N, W16), jnp.float32),
    ],
)


def _lin2_body(x_ref, p_ref, nd_ref, w_ref, b_ref, y_ref):
    agg = (p_ref[0, :, :D] + p_ref[1, :, :D]) * nd_ref[...][:, :1]
    y_ref[...] = (x_ref[...][:, D:]
                  + jnp.dot(agg, w_ref[...], preferred_element_type=jnp.float32)
                  + b_ref[...])


_lin2 = pl.pallas_call(
    _lin2_body,
    grid=(N // RB,),
    in_specs=[
        pl.BlockSpec((RB, 2 * D), lambda i: (i, 0)),
        pl.BlockSpec((NC, RB, DF), lambda i: (0, i, 0)),
        pl.BlockSpec((RB, W16), lambda i: (i, 0)),
        pl.BlockSpec((D, D), lambda i: (0, 0)),
        pl.BlockSpec((1, D), lambda i: (0, 0)),
    ],
    out_specs=pl.BlockSpec((RB, D), lambda i: (i, 0)),
    out_shape=jax.ShapeDtypeStruct((N, D), jnp.float32),
)


def kernel(x, edge_index, W1, b1, W2, b2):
    src4 = edge_index[0].reshape(NC, NS, NBLK, SPB, CH)
    dst4 = edge_index[1].reshape(NC, NS, NBLK, SPB, CH)
    ones_row = jnp.ones((CH, DF), jnp.float32)
    zeros_agg = jnp.zeros((CPR, DF), jnp.float32)

    degp = _dego_kernel(src4, ones_row, zeros_agg)
    ns16, h1 = _norm(degp, x)
    p1 = _agg_kernel(h1, src4, dst4, zeros_agg)
    y1, h2, nd16 = _lin1(x, p1, ns16, W1, b1.reshape(1, D))
    p2 = _agg_kernel(h2, src4, dst4, zeros_agg)
    y2 = _lin2(x, p2, nd16, W2, b2.reshape(1, D))
    return jnp.concatenate([y1, y2], axis=-1)


# speedup vs baseline: 7.8117x; 1.2066x over previous
import functools

import jax
import jax.numpy as jnp
from jax import lax
from jax.experimental import pallas as pl
from jax.experimental.pallas import tpu as pltpu
from jax.experimental.pallas import tpu_sc as plsc

N = 10000
E = 320000
D = 64
DF = 128
NC = 2
NS = 16
NW = NC * NS
EPW = E // NW
CH = 80
NSTEP = EPW // CH
NBLK = 5
SPB = NSTEP // NBLK
CPR = 632
TAIL = N - (NS - 1) * CPR
W16 = 16
RB = 1000

_sc_mesh = plsc.VectorSubcoreMesh(core_axis_name="c", subcore_axis_name="s")


def _coop_rows(s, fn):
    r0 = pl.multiple_of(s * CPR, 8)

    @pl.when(s < NS - 1)
    def _():
        fn(r0, CPR)

    @pl.when(s == NS - 1)
    def _():
        fn(r0, TAIL)


@functools.partial(
    pl.kernel,
    out_type=jax.ShapeDtypeStruct((NC, N, DF), jnp.float32),
    mesh=_sc_mesh,
    scratch_types=[
        pltpu.VMEM_SHARED((N, DF), jnp.float32),
        pltpu.VMEM((SPB, CH), jnp.int32),
        pltpu.VMEM((CH, DF), jnp.float32),
    ],
)
def _dego_kernel(src4, ones_h, zeros_h, out, dg, src_t, ones_v):
    c = lax.axis_index("c")
    s = lax.axis_index("s")

    _coop_rows(s, lambda r0, nr: pltpu.sync_copy(
        zeros_h.at[pl.ds(0, nr)], dg.at[pl.ds(r0, nr)]))
    pltpu.sync_copy(ones_h, ones_v)
    plsc.subcore_barrier()

    @pl.loop(0, NSTEP)
    def _(j):
        jj = lax.rem(j, SPB)
        blk = lax.div(j, SPB)

        @pl.when(jj == 0)
        def _():
            pltpu.sync_copy(src4.at[c].at[s].at[blk], src_t)

        pltpu.sync_copy(ones_v, dg.at[src_t.at[jj]], add=True)

    plsc.subcore_barrier()
    _coop_rows(s, lambda r0, nr: pltpu.sync_copy(
        dg.at[pl.ds(r0, nr)], out.at[c].at[pl.ds(r0, nr)]))


@functools.partial(
    pl.kernel,
    out_type=jax.ShapeDtypeStruct((NC, N, DF), jnp.float32),
    mesh=_sc_mesh,
    scratch_types=[
        pltpu.VMEM_SHARED((N, DF), jnp.float32),
        pltpu.VMEM((SPB, CH), jnp.int32),
        pltpu.VMEM((SPB, CH), jnp.int32),
        pltpu.VMEM((2, CH, DF), jnp.float32),
        pltpu.SemaphoreType.DMA,
    ],
)
def _agg_kernel(h_hbm, src4, dst4, zeros_h, out, agg, src_t, dst_t, rows, sem):
    c = lax.axis_index("c")
    s = lax.axis_index("s")

    _coop_rows(s, lambda r0, nr: pltpu.sync_copy(
        zeros_h.at[pl.ds(0, nr)], agg.at[pl.ds(r0, nr)]))
    plsc.subcore_barrier()

    @pl.loop(0, NBLK)
    def _(blk):
        pltpu.sync_copy(src4.at[c].at[s].at[blk], src_t)
        pltpu.sync_copy(dst4.at[c].at[s].at[blk], dst_t)
        pltpu.async_copy(h_hbm.at[src_t.at[0]], rows.at[0], sem)

        @pl.loop(0, SPB)
        def _(jj):
            buf = lax.rem(jj, 2)
            pltpu.make_async_copy(h_hbm.at[src_t.at[jj]], rows.at[buf],
                                  sem).wait()

            @pl.when(jj < SPB - 1)
            def _():
                pltpu.async_copy(h_hbm.at[src_t.at[jj + 1]],
                                 rows.at[lax.rem(jj + 1, 2)], sem)

            pltpu.sync_copy(rows.at[buf], agg.at[dst_t.at[jj]], add=True)

    plsc.subcore_barrier()
    _coop_rows(s, lambda r0, nr: pltpu.sync_copy(
        agg.at[pl.ds(r0, nr)], out.at[c].at[pl.ds(r0, nr)]))


def _norm_body(degp_ref, x_ref, ns_ref, h1_ref):
    deg_o = degp_ref[0, :, 0:1] + degp_ref[1, :, 0:1]
    ns = jnp.where(deg_o > 0, lax.rsqrt(jnp.maximum(deg_o, 1.0)), 0.0)
    ns_ref[...] = jnp.broadcast_to(ns, (RB, W16))
    h1 = x_ref[...][:, D:] * ns
    pad = jnp.concatenate(
        [jnp.ones((RB, 1), jnp.float32), jnp.zeros((RB, D - 1), jnp.float32)],
        axis=-1)
    h1_ref[...] = jnp.concatenate([h1, pad], axis=-1)


_norm = pl.pallas_call(
    _norm_body,
    grid=(N // RB,),
    in_specs=[
        pl.BlockSpec((NC, RB, DF), lambda i: (0, i, 0)),
        pl.BlockSpec((RB, 2 * D), lambda i: (i, 0)),
    ],
    out_specs=[
        pl.BlockSpec((RB, W16), lambda i: (i, 0)),
        pl.BlockSpec((RB, DF), lambda i: (i, 0)),
    ],
    out_shape=[
        jax.ShapeDtypeStruct((N, W16), jnp.float32),
        jax.ShapeDtypeStruct((N, DF), jnp.float32),
    ],
)


def _lin1_body(x_ref, p_ref, ns_ref, w_ref, b_ref, y_ref, h2_ref, nd_ref):
    deg_i = p_ref[0, :, D:D + 1] + p_ref[1, :, D:D + 1]
    nd = jnp.where(deg_i > 0, lax.rsqrt(jnp.maximum(deg_i, 1.0)), 0.0)
    nd_ref[...] = jnp.broadcast_to(nd, (RB, W16))
    agg = (p_ref[0, :, :D] + p_ref[1, :, :D]) * nd
    y = (x_ref[...][:, :D]
         + jnp.dot(agg, w_ref[...], preferred_element_type=jnp.float32)
         + b_ref[...])
    y_ref[...] = y
    h2 = y * ns_ref[...][:, :1]
    h2_ref[...] = jnp.concatenate([h2, jnp.zeros_like(h2)], axis=-1)


_lin1 = pl.pallas_call(
    _lin1_body,
    grid=(N // RB,),
    in_specs=[
        pl.BlockSpec((RB, 2 * D), lambda i: (i, 0)),
        pl.BlockSpec((NC, RB, DF), lambda i: (0, i, 0)),
        pl.BlockSpec((RB, W16), lambda i: (i, 0)),
        pl.BlockSpec((D, D), lambda i: (0, 0)),
        pl.BlockSpec((1, D), lambda i: (0, 0)),
    ],
    out_specs=[
        pl.BlockSpec((RB, D), lambda i: (i, 0)),
        pl.BlockSpec((RB, DF), lambda i: (i, 0)),
        pl.BlockSpec((RB, W16), lambda i: (i, 0)),
    ],
    out_shape=[
        jax.ShapeDtypeStruct((N, D), jnp.float32),
        jax.ShapeDtypeStruct((N, DF), jnp.float32),
        jax.ShapeDtypeStruct((N, W16), jnp.float32),
    ],
)


def _lin2_body(x_ref, p_ref, nd_ref, w_ref, b_ref, y_ref):
    agg = (p_ref[0, :, :D] + p_ref[1, :, :D]) * nd_ref[...][:, :1]
    y_ref[...] = (x_ref[...][:, D:]
                  + jnp.dot(agg, w_ref[...], preferred_element_type=jnp.float32)
                  + b_ref[...])


_lin2 = pl.pallas_call(
    _lin2_body,
    grid=(N // RB,),
    in_specs=[
        pl.BlockSpec((RB, 2 * D), lambda i: (i, 0)),
        pl.BlockSpec((NC, RB, DF), lambda i: (0, i, 0)),
        pl.BlockSpec((RB, W16), lambda i: (i, 0)),
        pl.BlockSpec((D, D), lambda i: (0, 0)),
        pl.BlockSpec((1, D), lambda i: (0, 0)),
    ],
    out_specs=pl.BlockSpec((RB, D), lambda i: (i, 0)),
    out_shape=jax.ShapeDtypeStruct((N, D), jnp.float32),
)


def kernel(x, edge_index, W1, b1, W2, b2):
    src4 = edge_index[0].reshape(NC, NS, NBLK, SPB, CH)
    dst4 = edge_index[1].reshape(NC, NS, NBLK, SPB, CH)
    ones_row = jnp.ones((CH, DF), jnp.float32)
    zeros_agg = jnp.zeros((CPR, DF), jnp.float32)

    degp = _dego_kernel(src4, ones_row, zeros_agg)
    ns16, h1 = _norm(degp, x)
    p1 = _agg_kernel(h1, src4, dst4, zeros_agg)
    y1, h2, nd16 = _lin1(x, p1, ns16, W1, b1.reshape(1, D))
    p2 = _agg_kernel(h2, src4, dst4, zeros_agg)
    y2 = _lin2(x, p2, nd16, W2, b2.reshape(1, D))
    return jnp.concatenate([y1, y2], axis=-1)


# speedup vs baseline: 10.1234x; 1.2959x over previous
import functools

import jax
import jax.numpy as jnp
from jax import lax
from jax.experimental import pallas as pl
from jax.experimental.pallas import tpu as pltpu
from jax.experimental.pallas import tpu_sc as plsc

N = 10000
E = 320000
D = 64
DF = 128
NC = 2
NS = 16
NW = NC * NS
EPW = E // NW
CH = 80
NSTEP = EPW // CH
NBLK = 5
SPB = NSTEP // NBLK
CPR = 632
TAIL = N - (NS - 1) * CPR
W16 = 16
RB = 1000

_sc_mesh = plsc.VectorSubcoreMesh(core_axis_name="c", subcore_axis_name="s")


def _coop_rows(s, fn):
    r0 = pl.multiple_of(s * CPR, 8)

    @pl.when(s < NS - 1)
    def _():
        fn(r0, CPR)

    @pl.when(s == NS - 1)
    def _():
        fn(r0, TAIL)


@functools.partial(
    pl.kernel,
    out_type=jax.ShapeDtypeStruct((NC, N, DF), jnp.float32),
    mesh=_sc_mesh,
    scratch_types=[
        pltpu.VMEM_SHARED((N, DF), jnp.float32),
        pltpu.VMEM((SPB, CH), jnp.int32),
        pltpu.VMEM((CH, DF), jnp.float32),
    ],
)
def _dego_kernel(src4, ones_h, zeros_h, out, dg, src_t, ones_v):
    c = lax.axis_index("c")
    s = lax.axis_index("s")

    _coop_rows(s, lambda r0, nr: pltpu.sync_copy(
        zeros_h.at[pl.ds(0, nr)], dg.at[pl.ds(r0, nr)]))
    pltpu.sync_copy(ones_h, ones_v)
    plsc.subcore_barrier()

    @pl.loop(0, NSTEP)
    def _(j):
        jj = lax.rem(j, SPB)
        blk = lax.div(j, SPB)

        @pl.when(jj == 0)
        def _():
            pltpu.sync_copy(src4.at[c].at[s].at[blk], src_t)

        pltpu.sync_copy(ones_v, dg.at[src_t.at[jj]], add=True)

    plsc.subcore_barrier()
    _coop_rows(s, lambda r0, nr: pltpu.sync_copy(
        dg.at[pl.ds(r0, nr)], out.at[c].at[pl.ds(r0, nr)]))


@functools.partial(
    pl.kernel,
    out_type=jax.ShapeDtypeStruct((NC, N, DF), jnp.float32),
    mesh=_sc_mesh,
    scratch_types=[
        pltpu.VMEM_SHARED((N, DF), jnp.float32),
        pltpu.VMEM((SPB, CH), jnp.int32),
        pltpu.VMEM((SPB, CH), jnp.int32),
        pltpu.VMEM((3, CH, DF), jnp.float32),
        pltpu.SemaphoreType.DMA,
    ],
)
def _agg_kernel(h_hbm, src4, dst4, zeros_h, out, agg, src_t, dst_t, rows, sem):
    c = lax.axis_index("c")
    s = lax.axis_index("s")

    _coop_rows(s, lambda r0, nr: pltpu.sync_copy(
        zeros_h.at[pl.ds(0, nr)], agg.at[pl.ds(r0, nr)]))
    plsc.subcore_barrier()

    @pl.loop(0, NBLK)
    def _(blk):
        pltpu.sync_copy(src4.at[c].at[s].at[blk], src_t)
        pltpu.sync_copy(dst4.at[c].at[s].at[blk], dst_t)
        pltpu.async_copy(h_hbm.at[src_t.at[0]], rows.at[0], sem)
        pltpu.async_copy(h_hbm.at[src_t.at[1]], rows.at[1], sem)

        @pl.loop(0, SPB)
        def _(jj):
            buf = lax.rem(jj, 3)
            pltpu.make_async_copy(h_hbm.at[src_t.at[jj]], rows.at[buf],
                                  sem).wait()

            @pl.when(jj < SPB - 2)
            def _():
                pltpu.async_copy(h_hbm.at[src_t.at[jj + 2]],
                                 rows.at[lax.rem(jj + 2, 3)], sem)

            pltpu.sync_copy(rows.at[buf], agg.at[dst_t.at[jj]], add=True)

    plsc.subcore_barrier()
    _coop_rows(s, lambda r0, nr: pltpu.sync_copy(
        agg.at[pl.ds(r0, nr)], out.at[c].at[pl.ds(r0, nr)]))


def _norm_body(degp_ref, x_ref, ns_ref, h1_ref):
    deg_o = degp_ref[0, :, 0:1] + degp_ref[1, :, 0:1]
    ns = jnp.where(deg_o > 0, lax.rsqrt(jnp.maximum(deg_o, 1.0)), 0.0)
    ns_ref[...] = jnp.broadcast_to(ns, (RB, W16))
    h1 = x_ref[...][:, D:] * ns
    pad = jnp.concatenate(
        [jnp.ones((RB, 1), jnp.float32), jnp.zeros((RB, D - 1), jnp.float32)],
        axis=-1)
    h1_ref[...] = jnp.concatenate([h1, pad], axis=-1)


_norm = pl.pallas_call(
    _norm_body,
    grid=(N // RB,),
    in_specs=[
        pl.BlockSpec((NC, RB, DF), lambda i: (0, i, 0)),
        pl.BlockSpec((RB, 2 * D), lambda i: (i, 0)),
    ],
    out_specs=[
        pl.BlockSpec((RB, W16), lambda i: (i, 0)),
        pl.BlockSpec((RB, DF), lambda i: (i, 0)),
    ],
    out_shape=[
        jax.ShapeDtypeStruct((N, W16), jnp.float32),
        jax.ShapeDtypeStruct((N, DF), jnp.float32),
    ],
)


def _lin1_body(x_ref, p_ref, ns_ref, w_ref, b_ref, y_ref, h2_ref, nd_ref):
    deg_i = p_ref[0, :, D:D + 1] + p_ref[1, :, D:D + 1]
    nd = jnp.where(deg_i > 0, lax.rsqrt(jnp.maximum(deg_i, 1.0)), 0.0)
    nd_ref[...] = jnp.broadcast_to(nd, (RB, W16))
    agg = (p_ref[0, :, :D] + p_ref[1, :, :D]) * nd
    y = (x_ref[...][:, :D]
         + jnp.dot(agg, w_ref[...], preferred_element_type=jnp.float32)
         + b_ref[...])
    y_ref[...] = y
    h2 = y * ns_ref[...][:, :1]
    h2_ref[...] = jnp.concatenate([h2, jnp.zeros_like(h2)], axis=-1)


_lin1 = pl.pallas_call(
    _lin1_body,
    grid=(N // RB,),
    in_specs=[
        pl.BlockSpec((RB, 2 * D), lambda i: (i, 0)),
        pl.BlockSpec((NC, RB, DF), lambda i: (0, i, 0)),
        pl.BlockSpec((RB, W16), lambda i: (i, 0)),
        pl.BlockSpec((D, D), lambda i: (0, 0)),
        pl.BlockSpec((1, D), lambda i: (0, 0)),
    ],
    out_specs=[
        pl.BlockSpec((RB, D), lambda i: (i, 0)),
        pl.BlockSpec((RB, DF), lambda i: (i, 0)),
        pl.BlockSpec((RB, W16), lambda i: (i, 0)),
    ],
    out_shape=[
        jax.ShapeDtypeStruct((N, D), jnp.float32),
        jax.ShapeDtypeStruct((N, DF), jnp.float32),
        jax.ShapeDtypeStruct((N, W16), jnp.float32),
    ],
)


def _lin2_body(x_ref, p_ref, nd_ref, w_ref, b_ref, y_ref):
    agg = (p_ref[0, :, :D] + p_ref[1, :, :D]) * nd_ref[...][:, :1]
    y_ref[...] = (x_ref[...][:, D:]
                  + jnp.dot(agg, w_ref[...], preferred_element_type=jnp.float32)
                  + b_ref[...])


_lin2 = pl.pallas_call(
    _lin2_body,
    grid=(N // RB,),
    in_specs=[
        pl.BlockSpec((RB, 2 * D), lambda i: (i, 0)),
        pl.BlockSpec((NC, RB, DF), lambda i: (0, i, 0)),
        pl.BlockSpec((RB, W16), lambda i: (i, 0)),
        pl.BlockSpec((D, D), lambda i: (0, 0)),
        pl.BlockSpec((1, D), lambda i: (0, 0)),
    ],
    out_specs=pl.BlockSpec((RB, D), lambda i: (i, 0)),
    out_shape=jax.ShapeDtypeStruct((N, D), jnp.float32),
)


def kernel(x, edge_index, W1, b1, W2, b2):
    src4 = edge_index[0].reshape(NC, NS, NBLK, SPB, CH)
    dst4 = edge_index[1].reshape(NC, NS, NBLK, SPB, CH)
    ones_row = jnp.ones((CH, DF), jnp.float32)
    zeros_agg = jnp.zeros((CPR, DF), jnp.float32)

    degp = _dego_kernel(src4, ones_row, zeros_agg)
    ns16, h1 = _norm(degp, x)
    p1 = _agg_kernel(h1, src4, dst4, zeros_agg)
    y1, h2, nd16 = _lin1(x, p1, ns16, W1, b1.reshape(1, D))
    p2 = _agg_kernel(h2, src4, dst4, zeros_agg)
    y2 = _lin2(x, p2, nd16, W2, b2.reshape(1, D))
    return jnp.concatenate([y1, y2], axis=-1)


# speedup vs baseline: 10.4227x; 1.0296x over previous
import functools

import jax
import jax.numpy as jnp
from jax import lax
from jax.experimental import pallas as pl
from jax.experimental.pallas import tpu as pltpu
from jax.experimental.pallas import tpu_sc as plsc

N = 10000
E = 320000
D = 64
DF = 128
NC = 2
NS = 16
NW = NC * NS
EPW = E // NW
CH = 80
NSTEP = EPW // CH
NBLK = 5
SPB = NSTEP // NBLK
CPR = 632
TAIL = N - (NS - 1) * CPR
W16 = 16
RB = 1000

_sc_mesh = plsc.VectorSubcoreMesh(core_axis_name="c", subcore_axis_name="s")


def _coop_rows(s, fn):
    r0 = pl.multiple_of(s * CPR, 8)

    @pl.when(s < NS - 1)
    def _():
        fn(r0, CPR)

    @pl.when(s == NS - 1)
    def _():
        fn(r0, TAIL)


@functools.partial(
    pl.kernel,
    out_type=jax.ShapeDtypeStruct((NC, N, DF), jnp.float32),
    mesh=_sc_mesh,
    scratch_types=[
        pltpu.VMEM_SHARED((N, DF), jnp.float32),
        pltpu.VMEM((SPB, CH), jnp.int32),
        pltpu.VMEM((CH, DF), jnp.float32),
    ],
)
def _dego_kernel(src4, ones_h, zeros_h, out, dg, src_t, ones_v):
    c = lax.axis_index("c")
    s = lax.axis_index("s")

    _coop_rows(s, lambda r0, nr: pltpu.sync_copy(
        zeros_h.at[pl.ds(0, nr)], dg.at[pl.ds(r0, nr)]))
    pltpu.sync_copy(ones_h, ones_v)
    plsc.subcore_barrier()

    @pl.loop(0, NSTEP)
    def _(j):
        jj = lax.rem(j, SPB)
        blk = lax.div(j, SPB)

        @pl.when(jj == 0)
        def _():
            pltpu.sync_copy(src4.at[c].at[s].at[blk], src_t)

        pltpu.sync_copy(ones_v, dg.at[src_t.at[jj]], add=True)

    plsc.subcore_barrier()
    _coop_rows(s, lambda r0, nr: pltpu.sync_copy(
        dg.at[pl.ds(r0, nr)], out.at[c].at[pl.ds(r0, nr)]))


@functools.partial(
    pl.kernel,
    out_type=jax.ShapeDtypeStruct((NC, N, DF), jnp.float32),
    mesh=_sc_mesh,
    scratch_types=[
        pltpu.VMEM_SHARED((N, DF), jnp.float32),
        pltpu.VMEM((SPB, CH), jnp.int32),
        pltpu.VMEM((SPB, CH), jnp.int32),
        pltpu.VMEM((4, CH, DF), jnp.float32),
        pltpu.SemaphoreType.DMA,
    ],
)
def _agg_kernel(h_hbm, src4, dst4, zeros_h, out, agg, src_t, dst_t, rows, sem):
    c = lax.axis_index("c")
    s = lax.axis_index("s")

    _coop_rows(s, lambda r0, nr: pltpu.sync_copy(
        zeros_h.at[pl.ds(0, nr)], agg.at[pl.ds(r0, nr)]))
    plsc.subcore_barrier()

    @pl.loop(0, NBLK)
    def _(blk):
        pltpu.sync_copy(src4.at[c].at[s].at[blk], src_t)
        pltpu.sync_copy(dst4.at[c].at[s].at[blk], dst_t)
        pltpu.async_copy(h_hbm.at[src_t.at[0]], rows.at[0], sem)
        pltpu.async_copy(h_hbm.at[src_t.at[1]], rows.at[1], sem)
        pltpu.async_copy(h_hbm.at[src_t.at[2]], rows.at[2], sem)

        @pl.loop(0, SPB)
        def _(jj):
            buf = lax.rem(jj, 4)
            pltpu.make_async_copy(h_hbm.at[src_t.at[jj]], rows.at[buf],
                                  sem).wait()

            @pl.when(jj < SPB - 3)
            def _():
                pltpu.async_copy(h_hbm.at[src_t.at[jj + 3]],
                                 rows.at[lax.rem(jj + 3, 4)], sem)

            pltpu.sync_copy(rows.at[buf], agg.at[dst_t.at[jj]], add=True)

    plsc.subcore_barrier()
    _coop_rows(s, lambda r0, nr: pltpu.sync_copy(
        agg.at[pl.ds(r0, nr)], out.at[c].at[pl.ds(r0, nr)]))


def _norm_body(degp_ref, x_ref, ns_ref, h1_ref):
    deg_o = degp_ref[0, :, 0:1] + degp_ref[1, :, 0:1]
    ns = jnp.where(deg_o > 0, lax.rsqrt(jnp.maximum(deg_o, 1.0)), 0.0)
    ns_ref[...] = jnp.broadcast_to(ns, (RB, W16))
    h1 = x_ref[...][:, D:] * ns
    pad = jnp.concatenate(
        [jnp.ones((RB, 1), jnp.float32), jnp.zeros((RB, D - 1), jnp.float32)],
        axis=-1)
    h1_ref[...] = jnp.concatenate([h1, pad], axis=-1)


_norm = pl.pallas_call(
    _norm_body,
    grid=(N // RB,),
    in_specs=[
        pl.BlockSpec((NC, RB, DF), lambda i: (0, i, 0)),
        pl.BlockSpec((RB, 2 * D), lambda i: (i, 0)),
    ],
    out_specs=[
        pl.BlockSpec((RB, W16), lambda i: (i, 0)),
        pl.BlockSpec((RB, DF), lambda i: (i, 0)),
    ],
    out_shape=[
        jax.ShapeDtypeStruct((N, W16), jnp.float32),
        jax.ShapeDtypeStruct((N, DF), jnp.float32),
    ],
)


def _lin1_body(x_ref, p_ref, ns_ref, w_ref, b_ref, y_ref, h2_ref, nd_ref):
    deg_i = p_ref[0, :, D:D + 1] + p_ref[1, :, D:D + 1]
    nd = jnp.where(deg_i > 0, lax.rsqrt(jnp.maximum(deg_i, 1.0)), 0.0)
    nd_ref[...] = jnp.broadcast_to(nd, (RB, W16))
    agg = (p_ref[0, :, :D] + p_ref[1, :, :D]) * nd
    y = (x_ref[...][:, :D]
         + jnp.dot(agg, w_ref[...], preferred_element_type=jnp.float32)
         + b_ref[...])
    y_ref[...] = y
    h2 = y * ns_ref[...][:, :1]
    h2_ref[...] = jnp.concatenate([h2, jnp.zeros_like(h2)], axis=-1)


_lin1 = pl.pallas_call(
    _lin1_body,
    grid=(N // RB,),
    in_specs=[
        pl.BlockSpec((RB, 2 * D), lambda i: (i, 0)),
        pl.BlockSpec((NC, RB, DF), lambda i: (0, i, 0)),
        pl.BlockSpec((RB, W16), lambda i: (i, 0)),
        pl.BlockSpec((D, D), lambda i: (0, 0)),
        pl.BlockSpec((1, D), lambda i: (0, 0)),
    ],
    out_specs=[
        pl.BlockSpec((RB, D), lambda i: (i, 0)),
        pl.BlockSpec((RB, DF), lambda i: (i, 0)),
        pl.BlockSpec((RB, W16), lambda i: (i, 0)),
    ],
    out_shape=[
        jax.ShapeDtypeStruct((N, D), jnp.float32),
        jax.ShapeDtypeStruct((N, DF), jnp.float32),
        jax.ShapeDtypeStruct((N, W16), jnp.float32),
    ],
)


def _lin2_body(x_ref, p_ref, nd_ref, w_ref, b_ref, y_ref):
    agg = (p_ref[0, :, :D] + p_ref[1, :, :D]) * nd_ref[...][:, :1]
    y_ref[...] = (x_ref[...][:, D:]
                  + jnp.dot(agg, w_ref[...], preferred_element_type=jnp.float32)
                  + b_ref[...])


_lin2 = pl.pallas_call(
    _lin2_body,
    grid=(N // RB,),
    in_specs=[
        pl.BlockSpec((RB, 2 * D), lambda i: (i, 0)),
        pl.BlockSpec((NC, RB, DF), lambda i: (0, i, 0)),
        pl.BlockSpec((RB, W16), lambda i: (i, 0)),
        pl.BlockSpec((D, D), lambda i: (0, 0)),
        pl.BlockSpec((1, D), lambda i: (0, 0)),
    ],
    out_specs=pl.BlockSpec((RB, D), lambda i: (i, 0)),
    out_shape=jax.ShapeDtypeStruct((N, D), jnp.float32),
)


def kernel(x, edge_index, W1, b1, W2, b2):
    src4 = edge_index[0].reshape(NC, NS, NBLK, SPB, CH)
    dst4 = edge_index[1].reshape(NC, NS, NBLK, SPB, CH)
    ones_row = jnp.ones((CH, DF), jnp.float32)
    zeros_agg = jnp.zeros((CPR, DF), jnp.float32)

    degp = _dego_kernel(src4, ones_row, zeros_agg)
    ns16, h1 = _norm(degp, x)
    p1 = _agg_kernel(h1, src4, dst4, zeros_agg)
    y1, h2, nd16 = _lin1(x, p1, ns16, W1, b1.reshape(1, D))
    p2 = _agg_kernel(h2, src4, dst4, zeros_agg)
    y2 = _lin2(x, p2, nd16, W2, b2.reshape(1, D))
    return jnp.concatenate([y1, y2], axis=-1)
